# baseline (speedup 1.0000x reference)
import functools
import jax
import jax.numpy as jnp
from jax import lax
from jax.experimental import pallas as pl
from jax.experimental.pallas import tpu as pltpu
from jax.experimental.pallas import tpu_sc as plsc

_W_SIZE = 8
_STRIDE = 4
_PAD = 2
_EXTRA = 2

_B, _C, _H, _W = 2, 128, 192, 192
_GRID = (_H + 2 * _PAD - _W_SIZE) // _STRIDE + 1
_M = 3000

_H0 = _H + 2 * _PAD
_W0P = 200
_PIX0 = _W_SIZE * _W_SIZE
_H1 = _H + 2 * (_PAD + _EXTRA)
_K1 = _W_SIZE + 2 * _EXTRA
_PIX1 = _K1 * _K1

_NW = 32
_NB0 = _M * _PIX0 // 128
_NB1 = _M * _PIX1 // 128
_BPW0 = 47
_BPW1 = 106
_IST0 = 48
_IST1 = 112
_NRING = 6


def _gather_kernel(f0t, f1t, idx0, idx1, out0, out1, i0_v, i1_v,
                   *bufs_and_sems):
    bufs = bufs_and_sems[:_NRING]
    gsems = bufs_and_sems[_NRING:2 * _NRING]
    wsems = bufs_and_sems[2 * _NRING:]
    wid = lax.axis_index("c") * 16 + lax.axis_index("s")
    last = wid == _NW - 1

    pltpu.sync_copy(idx0.at[pl.ds(wid * _IST0, _IST0)], i0_v)
    pltpu.sync_copy(idx1.at[pl.ds(wid * _IST1, _IST1)], i1_v)

    def run_pass(table, i_v, out, base, nb):
        nq = nb // _NRING

        def body(gq, carry):
            for k in range(_NRING):
                g = _NRING * gq + k

                @pl.when(gq > 0)
                def _(k=k):
                    pltpu.make_async_copy(bufs[k], out.at[base],
                                          wsems[k]).wait()

                pltpu.async_copy(table.at[i_v.at[g]], bufs[k], gsems[k])
            for k in range(_NRING):
                g = _NRING * gq + k
                pltpu.make_async_copy(table.at[i_v.at[g]], bufs[k],
                                      gsems[k]).wait()
                pltpu.async_copy(bufs[k], out.at[base + g], wsems[k])
            return carry

        lax.fori_loop(0, nq, body, 0)
        for k in range(_NRING):
            pltpu.make_async_copy(bufs[k], out.at[base], wsems[k]).wait()

        def tail(t, carry):
            g = _NRING * nq + t

            @pl.when(g < nb)
            def _():
                pltpu.async_copy(table.at[i_v.at[g]], bufs[0],
                                 gsems[0]).wait()
                pltpu.sync_copy(bufs[0], out.at[base + g])

            return carry

        lax.fori_loop(0, _NRING - 1, tail, 0)

    nb0 = jnp.where(last, _NB0 - (_NW - 1) * _BPW0, _BPW0)
    run_pass(f0t, i0_v, out0, wid * _BPW0, nb0)
    nb1 = jnp.where(last, _NB1 - (_NW - 1) * _BPW1, _BPW1)
    run_pass(f1t, i1_v, out1, wid * _BPW1, nb1)


def _pack_worker_idx(flat, per_worker, ist):
    flat = jnp.pad(flat, (0, _NW * per_worker - flat.shape[0]))
    flat = flat.reshape(_NW, per_worker)
    flat = jnp.pad(flat, ((0, 0), (0, ist * 128 - per_worker)))
    return flat.reshape(_NW * ist, 128)


@jax.jit
def kernel(feature0, feature1, b_idxes, i_idxes, j_idxes):
    f0t = jnp.pad(jnp.transpose(feature0, (0, 2, 3, 1)),
                  ((0, 0), (_PAD, _PAD), (_PAD, _W0P - _W - _PAD), (0, 0)))
    f0t = f0t.reshape(_B * _H0 * _W0P, _C)
    f1t = jnp.pad(jnp.transpose(feature1, (0, 2, 3, 1)),
                  ((0, 0), (_PAD + _EXTRA, _PAD + _EXTRA),
                   (_PAD + _EXTRA, _PAD + _EXTRA), (0, 0)))
    f1t = f1t.reshape(_B * _H1 * _H1, _C)

    b = b_idxes.astype(jnp.int32)
    i = i_idxes.astype(jnp.int32)
    j = j_idxes.astype(jnp.int32)

    r0 = (i // _GRID) * _STRIDE
    c0 = (i % _GRID) * _STRIDE
    p0 = jnp.arange(_PIX0, dtype=jnp.int32)
    idx0 = b[:, None] * (_H0 * _W0P) \
        + (r0[:, None] + p0[None, :] // _W_SIZE) * _W0P \
        + c0[:, None] + p0[None, :] % _W_SIZE
    idx0 = _pack_worker_idx(idx0.reshape(-1), _BPW0 * 128, _IST0)

    r1 = (j // _GRID) * _STRIDE
    c1 = (j % _GRID) * _STRIDE
    p1 = jnp.arange(_PIX1, dtype=jnp.int32)
    idx1 = b[:, None] * (_H1 * _H1) \
        + (r1[:, None] + p1[None, :] // _K1) * _H1 \
        + c1[:, None] + p1[None, :] % _K1
    idx1 = _pack_worker_idx(idx1.reshape(-1), _BPW1 * 128, _IST1)

    mesh = plsc.VectorSubcoreMesh(core_axis_name="c", subcore_axis_name="s")
    out0, out1 = pl.kernel(
        _gather_kernel,
        mesh=mesh,
        out_type=[
            jax.ShapeDtypeStruct((_NB0, 128, _C), jnp.float32),
            jax.ShapeDtypeStruct((_NB1, 128, _C), jnp.float32),
        ],
        scratch_types=[
            pltpu.VMEM((_IST0, 128), jnp.int32),
            pltpu.VMEM((_IST1, 128), jnp.int32),
        ]
        + [pltpu.VMEM((128, _C), jnp.float32)] * _NRING
        + [pltpu.SemaphoreType.DMA] * (2 * _NRING),
    )(f0t, f1t, idx0, idx1)

    fine0 = out0.reshape(_M, _PIX0, _C)
    fine1 = out1.reshape(_M, _PIX1, _C)
    return (fine0, fine1)


# speedup vs baseline: 1.0174x; 1.0174x over previous
import functools
import jax
import jax.numpy as jnp
from jax import lax
from jax.experimental import pallas as pl
from jax.experimental.pallas import tpu as pltpu
from jax.experimental.pallas import tpu_sc as plsc

_W_SIZE = 8
_STRIDE = 4
_PAD = 2
_EXTRA = 2

_B, _C, _H, _W = 2, 128, 192, 192
_GRID = (_H + 2 * _PAD - _W_SIZE) // _STRIDE + 1
_M = 3000

_H0 = _H + 2 * _PAD
_W0P = 200
_PIX0 = _W_SIZE * _W_SIZE
_H1 = _H + 2 * (_PAD + _EXTRA)
_K1 = _W_SIZE + 2 * _EXTRA
_PIX1 = _K1 * _K1

_NW = 32
_NB0 = _M * _PIX0 // 128
_NB1 = _M * _PIX1 // 128
_BPW0 = 48
_BPW1 = 108
_IST0 = 48
_IST1 = 112
_NRING = 6


def _gather_kernel(f0t, f1t, idx0, idx1, out0, out1, i0_v, i1_v,
                   *bufs_and_sems):
    bufs = bufs_and_sems[:_NRING]
    gsems = bufs_and_sems[_NRING:2 * _NRING]
    wsems = bufs_and_sems[2 * _NRING:]
    wid = lax.axis_index("c") * 16 + lax.axis_index("s")
    last = wid == _NW - 1

    pltpu.sync_copy(idx0.at[pl.ds(wid * _IST0, _IST0)], i0_v)
    pltpu.sync_copy(idx1.at[pl.ds(wid * _IST1, _IST1)], i1_v)

    def run_pass(table, i_v, out, base, nb):
        nq = nb // _NRING

        def body(gq, carry):
            for k in range(_NRING):
                g = _NRING * gq + k

                @pl.when(gq > 0)
                def _(k=k):
                    pltpu.make_async_copy(bufs[k], out.at[base],
                                          wsems[k]).wait()

                pltpu.async_copy(table.at[i_v.at[g]], bufs[k], gsems[k])
            for k in range(_NRING):
                g = _NRING * gq + k
                pltpu.make_async_copy(table.at[i_v.at[g]], bufs[k],
                                      gsems[k]).wait()
                pltpu.async_copy(bufs[k], out.at[base + g], wsems[k])
            return carry

        lax.fori_loop(0, nq, body, 0)
        for k in range(_NRING):
            pltpu.make_async_copy(bufs[k], out.at[base], wsems[k]).wait()

        def tail(t, carry):
            g = _NRING * nq + t

            @pl.when(g < nb)
            def _():
                pltpu.async_copy(table.at[i_v.at[g]], bufs[0],
                                 gsems[0]).wait()
                pltpu.sync_copy(bufs[0], out.at[base + g])

            return carry

        lax.fori_loop(0, _NRING - 1, tail, 0)

    nb0 = jnp.where(last, _NB0 - (_NW - 1) * _BPW0, _BPW0)
    run_pass(f0t, i0_v, out0, wid * _BPW0, nb0)
    nb1 = jnp.where(last, _NB1 - (_NW - 1) * _BPW1, _BPW1)
    run_pass(f1t, i1_v, out1, wid * _BPW1, nb1)


def _pack_worker_idx(flat, per_worker, ist):
    flat = jnp.pad(flat, (0, _NW * per_worker - flat.shape[0]))
    flat = flat.reshape(_NW, per_worker)
    flat = jnp.pad(flat, ((0, 0), (0, ist * 128 - per_worker)))
    return flat.reshape(_NW * ist, 128)


@jax.jit
def kernel(feature0, feature1, b_idxes, i_idxes, j_idxes):
    f0t = jnp.pad(jnp.transpose(feature0, (0, 2, 3, 1)),
                  ((0, 0), (_PAD, _PAD), (_PAD, _W0P - _W - _PAD), (0, 0)))
    f0t = f0t.reshape(_B * _H0 * _W0P, _C)
    f1t = jnp.pad(jnp.transpose(feature1, (0, 2, 3, 1)),
                  ((0, 0), (_PAD + _EXTRA, _PAD + _EXTRA),
                   (_PAD + _EXTRA, _PAD + _EXTRA), (0, 0)))
    f1t = f1t.reshape(_B * _H1 * _H1, _C)

    b = b_idxes.astype(jnp.int32)
    i = i_idxes.astype(jnp.int32)
    j = j_idxes.astype(jnp.int32)

    r0 = (i // _GRID) * _STRIDE
    c0 = (i % _GRID) * _STRIDE
    p0 = jnp.arange(_PIX0, dtype=jnp.int32)
    idx0 = b[:, None] * (_H0 * _W0P) \
        + (r0[:, None] + p0[None, :] // _W_SIZE) * _W0P \
        + c0[:, None] + p0[None, :] % _W_SIZE
    idx0 = _pack_worker_idx(idx0.reshape(-1), _BPW0 * 128, _IST0)

    r1 = (j // _GRID) * _STRIDE
    c1 = (j % _GRID) * _STRIDE
    p1 = jnp.arange(_PIX1, dtype=jnp.int32)
    idx1 = b[:, None] * (_H1 * _H1) \
        + (r1[:, None] + p1[None, :] // _K1) * _H1 \
        + c1[:, None] + p1[None, :] % _K1
    idx1 = _pack_worker_idx(idx1.reshape(-1), _BPW1 * 128, _IST1)

    mesh = plsc.VectorSubcoreMesh(core_axis_name="c", subcore_axis_name="s")
    out0, out1 = pl.kernel(
        _gather_kernel,
        mesh=mesh,
        out_type=[
            jax.ShapeDtypeStruct((_NB0, 128, _C), jnp.float32),
            jax.ShapeDtypeStruct((_NB1, 128, _C), jnp.float32),
        ],
        scratch_types=[
            pltpu.VMEM((_IST0, 128), jnp.int32),
            pltpu.VMEM((_IST1, 128), jnp.int32),
        ]
        + [pltpu.VMEM((128, _C), jnp.float32)] * _NRING
        + [pltpu.SemaphoreType.DMA] * (2 * _NRING),
    )(f0t, f1t, idx0, idx1)

    fine0 = out0.reshape(_M, _PIX0, _C)
    fine1 = out1.reshape(_M, _PIX1, _C)
    return (fine0, fine1)


# speedup vs baseline: 1.0190x; 1.0016x over previous
import jax
import jax.numpy as jnp
from jax import lax
from jax.experimental import pallas as pl
from jax.experimental.pallas import tpu as pltpu
from jax.experimental.pallas import tpu_sc as plsc

_W_SIZE = 8
_STRIDE = 4
_PAD = 2
_EXTRA = 2

_B, _C, _H, _W = 2, 128, 192, 192
_GRID = (_H + 2 * _PAD - _W_SIZE) // _STRIDE + 1
_M = 3000

_H0 = _H + 2 * _PAD
_W0P = 200
_PIX0 = _W_SIZE * _W_SIZE
_H1 = _H + 2 * (_PAD + _EXTRA)
_K1 = _W_SIZE + 2 * _EXTRA
_PIX1 = _K1 * _K1

_NW = 32
_NB0 = _M * _PIX0 // 128
_NB1 = _M * _PIX1 // 128
_BPW0 = 48
_BPW1 = 108
_IST0 = 48
_IST1 = 112
_NRING = 6


def _gather_kernel(f0t, f1t, idx0, idx1, out0, out1, i0_v, i1_v,
                   *bufs_and_sems):
    bufs = bufs_and_sems[:_NRING]
    gsems = bufs_and_sems[_NRING:2 * _NRING]
    wsems = bufs_and_sems[2 * _NRING:]
    wid = lax.axis_index("c") * 16 + lax.axis_index("s")
    last = wid == _NW - 1

    pltpu.sync_copy(idx0.at[pl.ds(wid * _IST0, _IST0)], i0_v)
    pltpu.sync_copy(idx1.at[pl.ds(wid * _IST1, _IST1)], i1_v)

    def run_pass(table, i_v, out, base, nb):
        nq = nb // _NRING

        def body(gq, carry):
            for k in range(_NRING):
                g = _NRING * gq + k

                @pl.when(gq > 0)
                def _(k=k):
                    pltpu.make_async_copy(bufs[k], out.at[base],
                                          wsems[k]).wait()

                pltpu.async_copy(table.at[i_v.at[g]], bufs[k], gsems[k])
            for k in range(_NRING):
                g = _NRING * gq + k
                pltpu.make_async_copy(table.at[i_v.at[g]], bufs[k],
                                      gsems[k]).wait()
                pltpu.async_copy(bufs[k], out.at[base + g], wsems[k])
            return carry

        lax.fori_loop(0, nq, body, 0)
        for k in range(_NRING):
            pltpu.make_async_copy(bufs[k], out.at[base], wsems[k]).wait()

        def tail(t, carry):
            g = _NRING * nq + t

            @pl.when(g < nb)
            def _():
                pltpu.async_copy(table.at[i_v.at[g]], bufs[0],
                                 gsems[0]).wait()
                pltpu.sync_copy(bufs[0], out.at[base + g])

            return carry

        lax.fori_loop(0, _NRING - 1, tail, 0)

    nb0 = jnp.where(last, _NB0 - (_NW - 1) * _BPW0, _BPW0)
    run_pass(f0t, i0_v, out0, wid * _BPW0, nb0)
    nb1 = jnp.where(last, _NB1 - (_NW - 1) * _BPW1, _BPW1)
    run_pass(f1t, i1_v, out1, wid * _BPW1, nb1)


def _pack_worker_idx(flat, per_worker, ist):
    flat = jnp.pad(flat, (0, _NW * per_worker - flat.shape[0]))
    flat = flat.reshape(_NW, per_worker)
    flat = jnp.pad(flat, ((0, 0), (0, ist * 128 - per_worker)))
    return flat.reshape(_NW * ist, 128)


@jax.jit
def kernel(feature0, feature1, b_idxes, i_idxes, j_idxes):
    f0t = jnp.pad(jnp.transpose(feature0, (0, 2, 3, 1)),
                  ((0, 0), (_PAD, _PAD), (_PAD, _W0P - _W - _PAD), (0, 0)))
    f0t = f0t.reshape(_B * _H0 * _W0P, _C)
    f1t = jnp.pad(jnp.transpose(feature1, (0, 2, 3, 1)),
                  ((0, 0), (_PAD + _EXTRA, _PAD + _EXTRA),
                   (_PAD + _EXTRA, _PAD + _EXTRA), (0, 0)))
    f1t = f1t.reshape(_B * _H1 * _H1, _C)

    b = b_idxes.astype(jnp.int32)
    i = i_idxes.astype(jnp.int32)
    j = j_idxes.astype(jnp.int32)

    r0 = (i // _GRID) * _STRIDE
    c0 = (i % _GRID) * _STRIDE
    p0 = jnp.arange(_PIX0, dtype=jnp.int32)
    idx0 = b[:, None] * (_H0 * _W0P) \
        + (r0[:, None] + p0[None, :] // _W_SIZE) * _W0P \
        + c0[:, None] + p0[None, :] % _W_SIZE
    idx0 = _pack_worker_idx(idx0.reshape(-1), _BPW0 * 128, _IST0)

    r1 = (j // _GRID) * _STRIDE
    c1 = (j % _GRID) * _STRIDE
    p1 = jnp.arange(_PIX1, dtype=jnp.int32)
    idx1 = b[:, None] * (_H1 * _H1) \
        + (r1[:, None] + p1[None, :] // _K1) * _H1 \
        + c1[:, None] + p1[None, :] % _K1
    idx1 = _pack_worker_idx(idx1.reshape(-1), _BPW1 * 128, _IST1)

    mesh = plsc.VectorSubcoreMesh(core_axis_name="c", subcore_axis_name="s")
    out0, out1 = pl.kernel(
        _gather_kernel,
        mesh=mesh,
        out_type=[
            jax.ShapeDtypeStruct((_NB0, 128, _C), jnp.float32),
            jax.ShapeDtypeStruct((_NB1, 128, _C), jnp.float32),
        ],
        scratch_types=[
            pltpu.VMEM((_IST0, 128), jnp.int32),
            pltpu.VMEM((_IST1, 128), jnp.int32),
        ]
        + [pltpu.VMEM((128, _C), jnp.float32)] * _NRING
        + [pltpu.SemaphoreType.DMA] * (2 * _NRING),
    )(f0t, f1t, idx0, idx1)

    fine0 = out0.reshape(_M, _PIX0, _C)
    fine1 = out1.reshape(_M, _PIX1, _C)
    return (fine0, fine1)
